# baseline (speedup 1.0000x reference)
import functools

import jax
import jax.numpy as jnp
from jax import lax
from jax.experimental import pallas as pl
from jax.experimental.pallas import tpu as pltpu
from jax.experimental.pallas import tpu_sc as plsc

_BLK = 256
_CHUNK = 32


def _routing(flat_ids, num_experts, blk):
    n = flat_ids.shape[0]
    oh = flat_ids[:, None] == jnp.arange(num_experts, dtype=jnp.int32)[None, :]
    csum = jnp.cumsum(oh.astype(jnp.int32), axis=0)
    rank = jnp.sum(jnp.where(oh, csum - 1, 0), axis=1)
    counts = csum[-1]
    padded = ((counts + blk - 1) // blk) * blk
    ends = jnp.cumsum(padded)
    off = ends - padded
    dst = rank + jnp.sum(jnp.where(oh, off[None, :], 0), axis=1)
    num_blocks = (n + num_experts * blk) // blk
    b_start = jnp.arange(num_blocks, dtype=jnp.int32) * blk
    block_expert = jnp.minimum(
        jnp.sum((b_start[:, None] >= ends[None, :]).astype(jnp.int32), axis=1),
        num_experts - 1).astype(jnp.int32)
    return dst.astype(jnp.int32), block_expert


def _sc_dispatch(x, tok_of_slot, dst, w16, np_rows):
    d = x.shape[1]
    info = plsc.get_sparse_core_info()
    nw = info.num_cores * info.num_subcores
    nch = tok_of_slot.shape[1]

    mesh = plsc.VectorSubcoreMesh(core_axis_name="c", subcore_axis_name="s")

    @functools.partial(
        pl.kernel, mesh=mesh,
        out_type=(jax.ShapeDtypeStruct((np_rows, d), jnp.float32),
                  jax.ShapeDtypeStruct((np_rows, 128), jnp.float32)),
        scratch_types=[
            pltpu.VMEM((nch, _CHUNK), jnp.int32),
            pltpu.VMEM((nch, _CHUNK), jnp.int32),
            pltpu.VMEM((nch, _CHUNK, 128), jnp.float32),
            pltpu.VMEM((3, _CHUNK, d), jnp.float32),
            pltpu.SemaphoreType.DMA((3,)),
            pltpu.SemaphoreType.DMA((3,)),
        ],
    )
    def disp(x_hbm, tok_hbm, dst_hbm, w16_hbm, xg_hbm, ws_hbm,
             tok_v, dst_v, w_v, rows_v, gsem, ssem):
        wid = lax.axis_index("s") * info.num_cores + lax.axis_index("c")
        pltpu.sync_copy(tok_hbm.at[wid], tok_v)
        pltpu.sync_copy(dst_hbm.at[wid], dst_v)
        pltpu.sync_copy(w16_hbm.at[wid], w_v)
        gathers = [None] * nch
        scatters = [None] * nch
        gathers[0] = pltpu.async_copy(
            x_hbm.at[tok_v.at[0]], rows_v.at[0], gsem.at[0])
        for c in range(nch):
            if c >= 2:
                scatters[c - 2].wait()
            if c + 1 < nch:
                b = (c + 1) % 3
                gathers[c + 1] = pltpu.async_copy(
                    x_hbm.at[tok_v.at[c + 1]], rows_v.at[b], gsem.at[b])
            gathers[c].wait()
            pltpu.sync_copy(w_v.at[c], ws_hbm.at[dst_v.at[c]])
            scatters[c] = pltpu.async_copy(
                rows_v.at[c % 3], xg_hbm.at[dst_v.at[c]], ssem.at[c % 3])
        scatters[nch - 2].wait()
        scatters[nch - 1].wait()

    return disp(x, tok_of_slot, dst, w16)


def _sc_combine(y, idx_w, num_tokens):
    d = y.shape[1]
    info = plsc.get_sparse_core_info()
    nw = info.num_cores * info.num_subcores
    kn = idx_w.shape[1]
    tch = idx_w.shape[2]
    nv = d // 16

    mesh = plsc.VectorSubcoreMesh(core_axis_name="c", subcore_axis_name="s")

    @functools.partial(
        pl.kernel, mesh=mesh,
        out_type=jax.ShapeDtypeStruct((num_tokens, d), jnp.float32),
        scratch_types=[
            pltpu.VMEM((kn, tch), jnp.int32),
            pltpu.VMEM((4, tch, d), jnp.float32),
            pltpu.VMEM((2, tch, d), jnp.float32),
            pltpu.SemaphoreType.DMA((4,)),
            pltpu.SemaphoreType.DMA((2,)),
        ],
    )
    def comb(y_hbm, idx_hbm, out_hbm, idx_v, rows_v, o_v, gsem, wsem):
        wid = lax.axis_index("s") * info.num_cores + lax.axis_index("c")
        nchk = kn // 2
        per_w = nchk * tch
        base = wid * per_w
        pltpu.sync_copy(idx_hbm.at[wid], idx_v)
        gath = {}

        def issue(c):
            for kk in range(2):
                b = (c % 2) * 2 + kk
                gath[(c, kk)] = pltpu.async_copy(
                    y_hbm.at[idx_v.at[kk * nchk + c]], rows_v.at[b],
                    gsem.at[b])

        issue(0)
        wr = [None, None]
        for c in range(nchk):
            if c + 1 < nchk:
                issue(c + 1)
            for kk in range(2):
                gath[(c, kk)].wait()
            if c >= 2:
                wr[c % 2].wait()
            ob = c % 2
            ra = (c % 2) * 2
            rb = ra + 1

            def tok_body(i, _):
                def vec_body(j, _):
                    off = pl.multiple_of(j * 16, 16)
                    av = rows_v[ra, i, pl.ds(off, 16)]
                    bv = rows_v[rb, i, pl.ds(off, 16)]
                    o_v[ob, i, pl.ds(off, 16)] = av + bv
                    return 0

                jax.lax.fori_loop(0, nv, vec_body, 0, unroll=8)
                return 0

            jax.lax.fori_loop(0, tch, tok_body, 0)
            wr[ob] = pltpu.async_copy(
                o_v.at[ob], out_hbm.at[pl.ds(base + c * tch, tch)],
                wsem.at[ob])
        wr[0].wait()
        wr[1].wait()

    return comb(y, idx_w)


def _gemm_body(dff, be_ref, xg_ref, w1_ref, w2_ref, ws_ref, y_ref):
    e = be_ref[pl.program_id(0)]
    x = xg_ref[...].astype(jnp.bfloat16)
    gate = jax.lax.dot_general(
        x, w1_ref[e, :dff, :], (((1,), (1,)), ((), ())),
        preferred_element_type=jnp.float32)
    up = jax.lax.dot_general(
        x, w1_ref[e, dff:, :], (((1,), (1,)), ((), ())),
        preferred_element_type=jnp.float32)
    act = (gate * jax.lax.logistic(gate) * up).astype(jnp.bfloat16)
    y_ref[...] = jax.lax.dot_general(
        act, w2_ref[e], (((1,), (1,)), ((), ())),
        preferred_element_type=jnp.float32) * ws_ref[:, 0:1]


def kernel(hidden_states, w1, w2, topk_weights, topk_ids):
    num_tokens, d = hidden_states.shape
    num_experts = w1.shape[0]
    dff = w2.shape[2]
    k = topk_ids.shape[1]
    n = num_tokens * k
    np_rows = n + num_experts * _BLK
    num_blocks = np_rows // _BLK

    info = plsc.get_sparse_core_info()
    nw = info.num_cores * info.num_subcores
    nch = n // (nw * _CHUNK)

    flat_ids = topk_ids.astype(jnp.int32).T.reshape(-1)
    dst, block_expert = _routing(flat_ids, num_experts, _BLK)
    dst3 = dst.reshape(nw, nch, _CHUNK)
    tok_of_slot = (jnp.arange(n, dtype=jnp.int32) % num_tokens).reshape(nw, nch, _CHUNK)

    tw_flat = topk_weights.T.reshape(-1)
    w16 = jnp.broadcast_to(tw_flat[:, None], (n, 128)).reshape(
        nw, nch, _CHUNK, 128)
    xg, ws = _sc_dispatch(hidden_states, tok_of_slot, dst3, w16, np_rows)

    y = pl.pallas_call(
        functools.partial(_gemm_body, dff),
        grid_spec=pltpu.PrefetchScalarGridSpec(
            num_scalar_prefetch=1,
            grid=(num_blocks,),
            in_specs=[
                pl.BlockSpec((_BLK, d), lambda b, be: (b, 0)),
                pl.BlockSpec((num_experts, 2 * dff, d), lambda b, be: (0, 0, 0)),
                pl.BlockSpec((num_experts, d, dff), lambda b, be: (0, 0, 0)),
                pl.BlockSpec((_BLK, 128), lambda b, be: (b, 0)),
            ],
            out_specs=pl.BlockSpec((_BLK, d), lambda b, be: (b, 0)),
        ),
        out_shape=jax.ShapeDtypeStruct((np_rows, d), jnp.float32),
    )(block_expert, xg, w1, w2, ws)

    tch = 16
    nchk = num_tokens // (nw * tch)
    idx_w = dst.reshape(k, nw, nchk, tch).swapaxes(0, 1).reshape(
        nw, k * nchk, tch)
    out = _sc_combine(y, idx_w, num_tokens)
    return out


# speedup vs baseline: 1.0568x; 1.0568x over previous
import functools

import jax
import jax.numpy as jnp
from jax import lax
from jax.experimental import pallas as pl
from jax.experimental.pallas import tpu as pltpu
from jax.experimental.pallas import tpu_sc as plsc

_BLK = 256
_CHUNK = 32


def _routing(flat_ids, num_experts, blk):
    n = flat_ids.shape[0]
    oh = flat_ids[:, None] == jnp.arange(num_experts, dtype=jnp.int32)[None, :]
    csum = jnp.cumsum(oh.astype(jnp.int32), axis=0)
    rank = jnp.sum(jnp.where(oh, csum - 1, 0), axis=1)
    counts = csum[-1]
    padded = ((counts + blk - 1) // blk) * blk
    ends = jnp.cumsum(padded)
    off = ends - padded
    dst = rank + jnp.sum(jnp.where(oh, off[None, :], 0), axis=1)
    num_blocks = (n + num_experts * blk) // blk
    b_start = jnp.arange(num_blocks, dtype=jnp.int32) * blk
    block_expert = jnp.minimum(
        jnp.sum((b_start[:, None] >= ends[None, :]).astype(jnp.int32), axis=1),
        num_experts - 1).astype(jnp.int32)
    return dst.astype(jnp.int32), block_expert


def _sc_dispatch(x, tok_of_slot, dst, np_rows):
    d = x.shape[1]
    info = plsc.get_sparse_core_info()
    nw = info.num_cores * info.num_subcores
    nch = tok_of_slot.shape[1]

    mesh = plsc.VectorSubcoreMesh(core_axis_name="c", subcore_axis_name="s")

    @functools.partial(
        pl.kernel, mesh=mesh,
        out_type=jax.ShapeDtypeStruct((np_rows, d), jnp.float32),
        scratch_types=[
            pltpu.VMEM((nch, _CHUNK), jnp.int32),
            pltpu.VMEM((nch, _CHUNK), jnp.int32),
            pltpu.VMEM((3, _CHUNK, d), jnp.float32),
            pltpu.SemaphoreType.DMA((3,)),
            pltpu.SemaphoreType.DMA((3,)),
        ],
    )
    def disp(x_hbm, tok_hbm, dst_hbm, xg_hbm, tok_v, dst_v, rows_v, gsem, ssem):
        wid = lax.axis_index("s") * info.num_cores + lax.axis_index("c")
        pltpu.sync_copy(tok_hbm.at[wid], tok_v)
        pltpu.sync_copy(dst_hbm.at[wid], dst_v)
        gathers = [None] * nch
        scatters = [None] * nch
        gathers[0] = pltpu.async_copy(
            x_hbm.at[tok_v.at[0]], rows_v.at[0], gsem.at[0])
        for c in range(nch):
            if c >= 2:
                scatters[c - 2].wait()
            if c + 1 < nch:
                b = (c + 1) % 3
                gathers[c + 1] = pltpu.async_copy(
                    x_hbm.at[tok_v.at[c + 1]], rows_v.at[b], gsem.at[b])
            gathers[c].wait()
            scatters[c] = pltpu.async_copy(
                rows_v.at[c % 3], xg_hbm.at[dst_v.at[c]], ssem.at[c % 3])
        scatters[nch - 2].wait()
        scatters[nch - 1].wait()

    return disp(x, tok_of_slot, dst)


def _sc_unpermute(y, dst, n):
    d = y.shape[1]
    info = plsc.get_sparse_core_info()
    nw = info.num_cores * info.num_subcores
    nch = dst.shape[1]
    per_w = nch * _CHUNK

    mesh = plsc.VectorSubcoreMesh(core_axis_name="c", subcore_axis_name="s")

    @functools.partial(
        pl.kernel, mesh=mesh,
        out_type=jax.ShapeDtypeStruct((n, d), jnp.float32),
        scratch_types=[
            pltpu.VMEM((nch, _CHUNK), jnp.int32),
            pltpu.VMEM((3, _CHUNK, d), jnp.float32),
            pltpu.SemaphoreType.DMA((3,)),
            pltpu.SemaphoreType.DMA((3,)),
        ],
    )
    def unperm(y_hbm, dst_hbm, yf_hbm, idx_v, rows_v, gsem, wsem):
        wid = lax.axis_index("s") * info.num_cores + lax.axis_index("c")
        base = wid * per_w
        pltpu.sync_copy(dst_hbm.at[wid], idx_v)
        gathers = [None] * nch
        writes = [None] * nch
        gathers[0] = pltpu.async_copy(
            y_hbm.at[idx_v.at[0]], rows_v.at[0], gsem.at[0])
        for c in range(nch):
            if c >= 2:
                writes[c - 2].wait()
            if c + 1 < nch:
                b = (c + 1) % 3
                gathers[c + 1] = pltpu.async_copy(
                    y_hbm.at[idx_v.at[c + 1]], rows_v.at[b], gsem.at[b])
            gathers[c].wait()
            writes[c] = pltpu.async_copy(
                rows_v.at[c % 3],
                yf_hbm.at[pl.ds(base + c * _CHUNK, _CHUNK)], wsem.at[c % 3])
        writes[nch - 2].wait()
        writes[nch - 1].wait()

    return unperm(y, dst)


def _gemm_body(dff, be_ref, xg_ref, w1_ref, w2_ref, y_ref):
    e = be_ref[pl.program_id(0)]
    x = xg_ref[...].astype(jnp.bfloat16)
    gate = jax.lax.dot_general(
        x, w1_ref[e, :dff, :], (((1,), (1,)), ((), ())),
        preferred_element_type=jnp.float32)
    up = jax.lax.dot_general(
        x, w1_ref[e, dff:, :], (((1,), (1,)), ((), ())),
        preferred_element_type=jnp.float32)
    act = (gate * jax.lax.logistic(gate) * up).astype(jnp.bfloat16)
    y_ref[...] = jax.lax.dot_general(
        act, w2_ref[e], (((1,), (1,)), ((), ())),
        preferred_element_type=jnp.float32)


def _combine_body(yf_ref, tw_ref, o_ref):
    kk = pl.program_id(1)
    tw = tw_ref[...]
    lane = jax.lax.broadcasted_iota(jnp.int32, tw.shape, 1)
    c = jnp.sum(jnp.where(lane == kk, tw, 0.0), axis=1, keepdims=True)
    contrib = c * yf_ref[...]

    @pl.when(kk == 0)
    def _init():
        o_ref[...] = contrib

    @pl.when(kk > 0)
    def _acc():
        o_ref[...] += contrib


def kernel(hidden_states, w1, w2, topk_weights, topk_ids):
    num_tokens, d = hidden_states.shape
    num_experts = w1.shape[0]
    dff = w2.shape[2]
    k = topk_ids.shape[1]
    n = num_tokens * k
    np_rows = n + num_experts * _BLK
    num_blocks = np_rows // _BLK

    info = plsc.get_sparse_core_info()
    nw = info.num_cores * info.num_subcores
    nch = n // (nw * _CHUNK)

    flat_ids = topk_ids.astype(jnp.int32).T.reshape(-1)
    dst, block_expert = _routing(flat_ids, num_experts, _BLK)
    dst3 = dst.reshape(nw, nch, _CHUNK)
    tok_of_slot = (jnp.arange(n, dtype=jnp.int32) % num_tokens).reshape(nw, nch, _CHUNK)

    xg = _sc_dispatch(hidden_states, tok_of_slot, dst3, np_rows)

    y = pl.pallas_call(
        functools.partial(_gemm_body, dff),
        grid_spec=pltpu.PrefetchScalarGridSpec(
            num_scalar_prefetch=1,
            grid=(num_blocks,),
            in_specs=[
                pl.BlockSpec((_BLK, d), lambda b, be: (b, 0)),
                pl.BlockSpec((num_experts, 2 * dff, d), lambda b, be: (0, 0, 0)),
                pl.BlockSpec((num_experts, d, dff), lambda b, be: (0, 0, 0)),
            ],
            out_specs=pl.BlockSpec((_BLK, d), lambda b, be: (b, 0)),
        ),
        out_shape=jax.ShapeDtypeStruct((np_rows, d), jnp.float32),
    )(block_expert, xg, w1, w2)

    yflat = _sc_unpermute(y, dst3, n)

    bt = 512
    out = pl.pallas_call(
        _combine_body,
        grid=(num_tokens // bt, k),
        in_specs=[
            pl.BlockSpec((bt, d), lambda t, kk: (kk * (num_tokens // bt) + t, 0)),
            pl.BlockSpec((bt, k), lambda t, kk: (t, 0)),
        ],
        out_specs=pl.BlockSpec((bt, d), lambda t, kk: (t, 0)),
        out_shape=jax.ShapeDtypeStruct((num_tokens, d), jnp.float32),
    )(yflat, topk_weights)
    return out


# speedup vs baseline: 1.0755x; 1.0178x over previous
import functools

import jax
import jax.numpy as jnp
from jax import lax
from jax.experimental import pallas as pl
from jax.experimental.pallas import tpu as pltpu
from jax.experimental.pallas import tpu_sc as plsc

_BLK = 256
_CHUNK = 32


def _routing(flat_ids, num_experts, blk):
    n = flat_ids.shape[0]
    oh = flat_ids[:, None] == jnp.arange(num_experts, dtype=jnp.int32)[None, :]
    csum = jnp.cumsum(oh.astype(jnp.int32), axis=0)
    rank = jnp.sum(jnp.where(oh, csum - 1, 0), axis=1)
    counts = csum[-1]
    padded = ((counts + blk - 1) // blk) * blk
    ends = jnp.cumsum(padded)
    off = ends - padded
    dst = rank + jnp.sum(jnp.where(oh, off[None, :], 0), axis=1)
    num_blocks = (n + num_experts * blk) // blk
    b_start = jnp.arange(num_blocks, dtype=jnp.int32) * blk
    block_expert = jnp.minimum(
        jnp.sum((b_start[:, None] >= ends[None, :]).astype(jnp.int32), axis=1),
        num_experts - 1).astype(jnp.int32)
    return dst.astype(jnp.int32), block_expert


def _sc_dispatch(x, tok_of_slot, dst, np_rows):
    d = x.shape[1]
    info = plsc.get_sparse_core_info()
    nw = info.num_cores * info.num_subcores
    nch = tok_of_slot.shape[1]

    mesh = plsc.VectorSubcoreMesh(core_axis_name="c", subcore_axis_name="s")

    @functools.partial(
        pl.kernel, mesh=mesh,
        out_type=jax.ShapeDtypeStruct((np_rows, d), jnp.float32),
        scratch_types=[
            pltpu.VMEM((nch, _CHUNK), jnp.int32),
            pltpu.VMEM((nch, _CHUNK), jnp.int32),
            pltpu.VMEM((3, _CHUNK, d), jnp.float32),
            pltpu.SemaphoreType.DMA((3,)),
            pltpu.SemaphoreType.DMA((3,)),
        ],
    )
    def disp(x_hbm, tok_hbm, dst_hbm, xg_hbm, tok_v, dst_v, rows_v, gsem, ssem):
        wid = lax.axis_index("s") * info.num_cores + lax.axis_index("c")
        pltpu.sync_copy(tok_hbm.at[wid], tok_v)
        pltpu.sync_copy(dst_hbm.at[wid], dst_v)
        gathers = [None] * nch
        scatters = [None] * nch
        gathers[0] = pltpu.async_copy(
            x_hbm.at[tok_v.at[0]], rows_v.at[0], gsem.at[0])
        for c in range(nch):
            if c >= 2:
                scatters[c - 2].wait()
            if c + 1 < nch:
                b = (c + 1) % 3
                gathers[c + 1] = pltpu.async_copy(
                    x_hbm.at[tok_v.at[c + 1]], rows_v.at[b], gsem.at[b])
            gathers[c].wait()
            scatters[c] = pltpu.async_copy(
                rows_v.at[c % 3], xg_hbm.at[dst_v.at[c]], ssem.at[c % 3])
        scatters[nch - 2].wait()
        scatters[nch - 1].wait()

    return disp(x, tok_of_slot, dst)


def _sc_unpermute(y, dst, n):
    d = y.shape[1]
    info = plsc.get_sparse_core_info()
    nw = info.num_cores * info.num_subcores
    nch = dst.shape[1]
    per_w = nch * _CHUNK

    mesh = plsc.VectorSubcoreMesh(core_axis_name="c", subcore_axis_name="s")

    @functools.partial(
        pl.kernel, mesh=mesh,
        out_type=jax.ShapeDtypeStruct((n, d), jnp.float32),
        scratch_types=[
            pltpu.VMEM((nch, _CHUNK), jnp.int32),
            pltpu.VMEM((3, _CHUNK, d), jnp.float32),
            pltpu.SemaphoreType.DMA((3,)),
            pltpu.SemaphoreType.DMA((3,)),
        ],
    )
    def unperm(y_hbm, dst_hbm, yf_hbm, idx_v, rows_v, gsem, wsem):
        wid = lax.axis_index("s") * info.num_cores + lax.axis_index("c")
        base = wid * per_w
        pltpu.sync_copy(dst_hbm.at[wid], idx_v)
        gathers = [None] * nch
        writes = [None] * nch
        gathers[0] = pltpu.async_copy(
            y_hbm.at[idx_v.at[0]], rows_v.at[0], gsem.at[0])
        for c in range(nch):
            if c >= 2:
                writes[c - 2].wait()
            if c + 1 < nch:
                b = (c + 1) % 3
                gathers[c + 1] = pltpu.async_copy(
                    y_hbm.at[idx_v.at[c + 1]], rows_v.at[b], gsem.at[b])
            gathers[c].wait()
            writes[c] = pltpu.async_copy(
                rows_v.at[c % 3],
                yf_hbm.at[pl.ds(base + c * _CHUNK, _CHUNK)], wsem.at[c % 3])
        writes[nch - 2].wait()
        writes[nch - 1].wait()

    return unperm(y, dst)


def _gemm_body(dff, be_ref, xg_ref, w1_ref, w2_ref, y_ref):
    e = be_ref[pl.program_id(0)]
    x = xg_ref[...].astype(jnp.bfloat16)
    gate = jax.lax.dot_general(
        x, w1_ref[e, :dff, :], (((1,), (1,)), ((), ())),
        preferred_element_type=jnp.float32)
    up = jax.lax.dot_general(
        x, w1_ref[e, dff:, :], (((1,), (1,)), ((), ())),
        preferred_element_type=jnp.float32)
    act = (gate * jax.lax.logistic(gate) * up).astype(jnp.bfloat16)
    y_ref[...] = jax.lax.dot_general(
        act, w2_ref[e], (((1,), (1,)), ((), ())),
        preferred_element_type=jnp.float32)


def _combine_body(yf_ref, tw_ref, o_ref):
    kk = pl.program_id(1)
    tw = tw_ref[...]
    lane = jax.lax.broadcasted_iota(jnp.int32, tw.shape, 1)
    c = jnp.sum(jnp.where(lane == kk, tw, 0.0), axis=1, keepdims=True)
    contrib = c * yf_ref[...]

    @pl.when(kk == 0)
    def _init():
        o_ref[...] = contrib

    @pl.when(kk > 0)
    def _acc():
        o_ref[...] += contrib


def kernel(hidden_states, w1, w2, topk_weights, topk_ids):
    num_tokens, d = hidden_states.shape
    num_experts = w1.shape[0]
    dff = w2.shape[2]
    k = topk_ids.shape[1]
    n = num_tokens * k
    np_rows = n + num_experts * _BLK
    num_blocks = np_rows // _BLK

    info = plsc.get_sparse_core_info()
    nw = info.num_cores * info.num_subcores
    nch = n // (nw * _CHUNK)

    flat_ids = topk_ids.astype(jnp.int32).T.reshape(-1)
    dst, block_expert = _routing(flat_ids, num_experts, _BLK)
    dst3 = dst.reshape(nw, nch, _CHUNK)
    tok_of_slot = (jnp.arange(n, dtype=jnp.int32) % num_tokens).reshape(nw, nch, _CHUNK)

    xg = _sc_dispatch(hidden_states, tok_of_slot, dst3, np_rows)

    y = pl.pallas_call(
        functools.partial(_gemm_body, dff),
        grid_spec=pltpu.PrefetchScalarGridSpec(
            num_scalar_prefetch=1,
            grid=(num_blocks,),
            in_specs=[
                pl.BlockSpec((_BLK, d), lambda b, be: (b, 0)),
                pl.BlockSpec((num_experts, 2 * dff, d), lambda b, be: (0, 0, 0)),
                pl.BlockSpec((num_experts, d, dff), lambda b, be: (0, 0, 0)),
            ],
            out_specs=pl.BlockSpec((_BLK, d), lambda b, be: (b, 0)),
        ),
        out_shape=jax.ShapeDtypeStruct((np_rows, d), jnp.float32),
    )(block_expert, xg, w1, w2)

    yflat = _sc_unpermute(y, dst3, n)

    bt = 1024
    out = pl.pallas_call(
        _combine_body,
        grid=(num_tokens // bt, k),
        in_specs=[
            pl.BlockSpec((bt, d), lambda t, kk: (kk * (num_tokens // bt) + t, 0)),
            pl.BlockSpec((bt, k), lambda t, kk: (t, 0)),
        ],
        out_specs=pl.BlockSpec((bt, d), lambda t, kk: (t, 0)),
        out_shape=jax.ShapeDtypeStruct((num_tokens, d), jnp.float32),
    )(yflat, topk_weights)
    return out
